# baseline (speedup 1.0000x reference)
import functools

import jax
import jax.numpy as jnp
from jax import lax
from jax.experimental import pallas as pl
from jax.experimental.pallas import tpu as pltpu, tpu_sc as plsc

EMBD = 128
B = 16384
NC = 2
NS = 16
NW = NC * NS
ROWS_PER_W = B // NW
L = 16
NVEC = EMBD // L

_mesh = plsc.VectorSubcoreMesh(core_axis_name="c", subcore_axis_name="s")


@functools.partial(
    pl.kernel,
    mesh=_mesh,
    out_type=jax.ShapeDtypeStruct((B, EMBD), jnp.float32),
    scratch_types=[
        pltpu.VMEM((ROWS_PER_W, EMBD), jnp.float32),
        pltpu.VMEM((ROWS_PER_W,), jnp.float32),
        pltpu.VMEM((EMBD,), jnp.float32),
        pltpu.VMEM((EMBD,), jnp.float32),
    ],
)
def _depie_sc(user_hbm, td_hbm, w_hbm, b_hbm, out_hbm, slab, td_v, w_v, b_v):
    wid = lax.axis_index("s") * NC + lax.axis_index("c")
    base = wid * ROWS_PER_W

    pltpu.sync_copy(user_hbm.at[pl.ds(base, ROWS_PER_W)], slab)
    pltpu.sync_copy(td_hbm.at[pl.ds(base, ROWS_PER_W)], td_v)
    pltpu.sync_copy(w_hbm, w_v)
    pltpu.sync_copy(b_hbm, b_v)

    ws = [w_v[pl.ds(j * L, L)] for j in range(NVEC)]
    obs = [b_v[pl.ds(j * L, L)] + 1.0 for j in range(NVEC)]

    def body(g, carry):
        tvec = td_v[pl.ds(g * L, L)]
        for r in range(L):
            i = g * L + r
            t = tvec[r]
            for j in range(NVEC):
                coef = ws[j] * t + obs[j]
                slab[i, pl.ds(j * L, L)] = slab[i, pl.ds(j * L, L)] * coef
        return carry

    lax.fori_loop(0, ROWS_PER_W // L, body, 0)

    pltpu.sync_copy(slab, out_hbm.at[pl.ds(base, ROWS_PER_W)])


def kernel(user_embd, item_embd, timediffs, W_embd, b_embd):
    del item_embd
    td = timediffs.reshape(B)
    w = W_embd.reshape(EMBD)
    return _depie_sc(user_embd, td, w, b_embd)


# speedup vs baseline: 1.0433x; 1.0433x over previous
import functools

import jax
import jax.numpy as jnp
from jax import lax
from jax.experimental import pallas as pl
from jax.experimental.pallas import tpu as pltpu, tpu_sc as plsc

EMBD = 128
B = 16384
NC = 2
NS = 16
NW = NC * NS
ROWS_PER_W = B // NW
L = 16
NVEC = EMBD // L
NCH = 4
CH = ROWS_PER_W // NCH

_mesh = plsc.VectorSubcoreMesh(core_axis_name="c", subcore_axis_name="s")


@functools.partial(
    pl.kernel,
    mesh=_mesh,
    out_type=jax.ShapeDtypeStruct((B, EMBD), jnp.float32),
    scratch_types=(
        [pltpu.VMEM((CH, EMBD), jnp.float32) for _ in range(NCH)]
        + [
            pltpu.VMEM((ROWS_PER_W,), jnp.float32),
            pltpu.VMEM((EMBD,), jnp.float32),
            pltpu.VMEM((EMBD,), jnp.float32),
        ]
        + [pltpu.SemaphoreType.DMA for _ in range(2 * NCH + 1)]
    ),
)
def _depie_sc(user_hbm, td_hbm, w_hbm, b_hbm, out_hbm, *scratch):
    bufs = scratch[:NCH]
    td_v, w_v, b_v = scratch[NCH:NCH + 3]
    isems = scratch[NCH + 3:2 * NCH + 3]
    osems = scratch[2 * NCH + 3:3 * NCH + 3]
    ssem = scratch[3 * NCH + 3]

    wid = lax.axis_index("s") * NC + lax.axis_index("c")
    base = wid * ROWS_PER_W

    ins = [
        pltpu.async_copy(
            user_hbm.at[pl.ds(base + c * CH, CH)], bufs[c], isems[c]
        )
        for c in range(NCH)
    ]
    cp_td = pltpu.async_copy(td_hbm.at[pl.ds(base, ROWS_PER_W)], td_v, ssem)
    cp_w = pltpu.async_copy(w_hbm, w_v, ssem)
    cp_b = pltpu.async_copy(b_hbm, b_v, ssem)
    cp_td.wait()
    cp_w.wait()
    cp_b.wait()

    ws = [w_v[pl.ds(j * L, L)] for j in range(NVEC)]
    obs = [b_v[pl.ds(j * L, L)] + 1.0 for j in range(NVEC)]

    outs = []
    for c in range(NCH):
        ins[c].wait()
        buf = bufs[c]

        def body(g, carry, c=c, buf=buf):
            tvec = td_v[pl.ds(c * CH + g * L, L)]
            for r in range(L):
                i = g * L + r
                t = tvec[r]
                for j in range(NVEC):
                    coef = ws[j] * t + obs[j]
                    buf[i, pl.ds(j * L, L)] = buf[i, pl.ds(j * L, L)] * coef
            return carry

        lax.fori_loop(0, CH // L, body, 0)
        outs.append(
            pltpu.async_copy(
                buf, out_hbm.at[pl.ds(base + c * CH, CH)], osems[c]
            )
        )
    for cp in outs:
        cp.wait()


def kernel(user_embd, item_embd, timediffs, W_embd, b_embd):
    del item_embd
    td = timediffs.reshape(B)
    w = W_embd.reshape(EMBD)
    return _depie_sc(user_embd, td, w, b_embd)


# speedup vs baseline: 1.3159x; 1.2613x over previous
import functools

import jax
import jax.numpy as jnp
from jax.experimental import pallas as pl
from jax.experimental.pallas import tpu as pltpu

EMBD = 128
B = 16384
BLOCK_ROWS = 1024


def _depie_body(u_ref, t_ref, w_ref, b_ref, o_ref):
    coef = t_ref[...] * w_ref[...] + b_ref[...]
    o_ref[...] = u_ref[...] * coef


@jax.jit
def _depie_tc(user_embd, timediffs, w_row, b_row):
    grid = (B // BLOCK_ROWS,)
    return pl.pallas_call(
        _depie_body,
        grid=grid,
        in_specs=[
            pl.BlockSpec((BLOCK_ROWS, EMBD), lambda i: (i, 0)),
            pl.BlockSpec((BLOCK_ROWS, 1), lambda i: (i, 0)),
            pl.BlockSpec((1, EMBD), lambda i: (0, 0)),
            pl.BlockSpec((1, EMBD), lambda i: (0, 0)),
        ],
        out_specs=pl.BlockSpec((BLOCK_ROWS, EMBD), lambda i: (i, 0)),
        out_shape=jax.ShapeDtypeStruct((B, EMBD), jnp.float32),
        compiler_params=pltpu.CompilerParams(
            dimension_semantics=("arbitrary",),
        ),
    )(user_embd, timediffs, w_row, b_row)


def kernel(user_embd, item_embd, timediffs, W_embd, b_embd):
    del item_embd
    w_row = W_embd.reshape(1, EMBD)
    b_row = (1.0 + b_embd).reshape(1, EMBD)
    return _depie_tc(user_embd, timediffs, w_row, b_row)


# speedup vs baseline: 1.6759x; 1.2736x over previous
import jax
import jax.numpy as jnp
from jax.experimental import pallas as pl
from jax.experimental.pallas import tpu as pltpu

EMBD = 128
B = 16384
R8 = B // 8
BLOCK_G = 128


def _depie_body(u_ref, t_ref, w_ref, b_ref, o_ref):
    t = t_ref[...][:, :, None]
    coef = t * w_ref[...] + (b_ref[...] + 1.0)
    o_ref[...] = u_ref[...] * coef


@jax.jit
def _depie_tc(user3, td2, w3, b3):
    grid = (R8 // BLOCK_G,)
    return pl.pallas_call(
        _depie_body,
        grid=grid,
        in_specs=[
            pl.BlockSpec((BLOCK_G, 8, EMBD), lambda i: (i, 0, 0)),
            pl.BlockSpec((BLOCK_G, 8), lambda i: (i, 0)),
            pl.BlockSpec((1, 1, EMBD), lambda i: (0, 0, 0)),
            pl.BlockSpec((1, 1, EMBD), lambda i: (0, 0, 0)),
        ],
        out_specs=pl.BlockSpec((BLOCK_G, 8, EMBD), lambda i: (i, 0, 0)),
        out_shape=jax.ShapeDtypeStruct((R8, 8, EMBD), jnp.float32),
        compiler_params=pltpu.CompilerParams(
            dimension_semantics=("parallel",),
        ),
    )(user3, td2, w3, b3)


def kernel(user_embd, item_embd, timediffs, W_embd, b_embd):
    del item_embd
    user3 = user_embd.reshape(R8, 8, EMBD)
    td2 = timediffs.reshape(R8, 8)
    w3 = W_embd.reshape(1, 1, EMBD)
    b3 = b_embd.reshape(1, 1, EMBD)
    out3 = _depie_tc(user3, td2, w3, b3)
    return out3.reshape(B, EMBD)


# speedup vs baseline: 1.9193x; 1.1452x over previous
import jax
import jax.numpy as jnp
from jax.experimental import pallas as pl
from jax.experimental.pallas import tpu as pltpu

EMBD = 128
B = 16384
R8 = B // 8
BLOCK_G = 512


def _depie_body(u_ref, t_ref, w_ref, b_ref, o_ref):
    t = t_ref[...].reshape(BLOCK_G, 8, 1)
    coef = t * w_ref[...] + (b_ref[...] + 1.0)
    o_ref[...] = u_ref[...] * coef


@jax.jit
def _depie_tc(user3, timediffs, w3, b3):
    grid = (R8 // BLOCK_G,)
    return pl.pallas_call(
        _depie_body,
        grid=grid,
        in_specs=[
            pl.BlockSpec((BLOCK_G, 8, EMBD), lambda i: (i, 0, 0)),
            pl.BlockSpec((BLOCK_G * 8, 1), lambda i: (i, 0)),
            pl.BlockSpec((1, 1, EMBD), lambda i: (0, 0, 0)),
            pl.BlockSpec((1, 1, EMBD), lambda i: (0, 0, 0)),
        ],
        out_specs=pl.BlockSpec((BLOCK_G, 8, EMBD), lambda i: (i, 0, 0)),
        out_shape=jax.ShapeDtypeStruct((R8, 8, EMBD), jnp.float32),
        compiler_params=pltpu.CompilerParams(
            dimension_semantics=("arbitrary",),
        ),
    )(user3, timediffs, w3, b3)


def kernel(user_embd, item_embd, timediffs, W_embd, b_embd):
    del item_embd
    user3 = user_embd.reshape(R8, 8, EMBD)
    w3 = W_embd.reshape(1, 1, EMBD)
    b3 = b_embd.reshape(1, 1, EMBD)
    out3 = _depie_tc(user3, timediffs, w3, b3)
    return out3.reshape(B, EMBD)
